# baseline (speedup 1.0000x reference)
import functools

import jax
import jax.numpy as jnp
from jax import lax
from jax.experimental import pallas as pl
from jax.experimental.pallas import tpu as pltpu
from jax.experimental.pallas import tpu_sc as plsc

B = 1024
D = 64
TOKEN_NUM = 1000000
X_MAX = 100.0
ALPHA = 0.75

NC = 2
NS = 16
NW = NC * NS
BPW = B // NW


def _sc_gather_one(idx_hbm, emb_hbm, rows_out, idx_v, rows_v, sem):
    wid = lax.axis_index("s") * NC + lax.axis_index("c")
    base = wid * BPW
    pltpu.sync_copy(idx_hbm.at[pl.ds(base, BPW)], idx_v)
    pltpu.async_copy(emb_hbm.at[idx_v], rows_v, sem).wait()
    pltpu.sync_copy(rows_v, rows_out.at[pl.ds(base, BPW)])


def _make_sc_kernel():
    return functools.partial(
        pl.kernel,
        out_type=jax.ShapeDtypeStruct((B, D), jnp.float32),
        mesh=plsc.VectorSubcoreMesh(core_axis_name="c", subcore_axis_name="s"),
        compiler_params=pltpu.CompilerParams(
            needs_layout_passes=False, use_tc_tiling_on_sc=False),
        scratch_types=[
            pltpu.VMEM((BPW,), jnp.int32),
            pltpu.VMEM((BPW, D), jnp.float32),
            pltpu.SemaphoreType.DMA,
        ],
    )(_sc_gather_one)


_sc_kernel_i = _make_sc_kernel()
_sc_kernel_j = _make_sc_kernel()


ROW_BLK = 128


def _tc_outer(xij_ref, ri_ref, rj_ref, bsum_ref, out_ref):
    xf = xij_ref[...].astype(jnp.float32)
    t = jnp.log(xf)
    fx = jnp.where(xf >= X_MAX, jnp.float32(1.0),
                   jnp.exp(ALPHA * jnp.log(xf * (1.0 / X_MAX))))
    dots = jnp.sum(ri_ref[...] * rj_ref[...], axis=1, keepdims=True)
    s = dots + bsum_ref[...]
    diff = s - t
    res = fx * diff * diff
    out_ref[...] = res[:, None, :]


def kernel(x, emb_i, emb_j, bi, bj):
    idx_i = x[:, 0]
    idx_j = x[:, 1]
    xij2 = x[:, 2].reshape(1, B)

    rows_i = _sc_kernel_i(idx_i, emb_i)
    rows_j = _sc_kernel_j(idx_j, emb_j)
    bsum = jnp.zeros((B, 1), jnp.float32)

    out2 = pl.pallas_call(
        _tc_outer,
        grid=(B // ROW_BLK,),
        in_specs=[
            pl.BlockSpec((1, B), lambda i: (0, 0)),
            pl.BlockSpec((ROW_BLK, D), lambda i: (i, 0)),
            pl.BlockSpec((ROW_BLK, D), lambda i: (i, 0)),
            pl.BlockSpec((ROW_BLK, 1), lambda i: (i, 0)),
        ],
        out_specs=pl.BlockSpec((ROW_BLK, 1, B), lambda i: (i, 0, 0)),
        out_shape=jax.ShapeDtypeStruct((B, 1, B), jnp.float32),
    )(xij2, rows_i, rows_j, bsum)

    return out2


# speedup vs baseline: 17.8160x; 17.8160x over previous
import functools

import jax
import jax.numpy as jnp
from jax import lax
from jax.experimental import pallas as pl
from jax.experimental.pallas import tpu as pltpu
from jax.experimental.pallas import tpu_sc as plsc

B = 1024
D = 64
TOKEN_NUM = 1000000
X_MAX = 100.0
ALPHA = 0.75

NC = 2
NS = 16
NW = NC * NS
BPW = B // NW
PK = 128
NBUF = 4


def _sc_gather(ii_hbm, ij_hbm, embT_i, embT_j, biT, bjT,
               packed_out,
               ii_v, ij_v, si_v, sj_v, sbi_v, sbj_v, p_v, sem):
    wid = lax.axis_index("s") * NC + lax.axis_index("c")
    base = wid * BPW
    chunk = pl.multiple_of((base // 128) * 128, 128)
    off = base - chunk
    pltpu.sync_copy(ii_hbm.at[pl.ds(chunk, 128)], ii_v)
    pltpu.sync_copy(ij_hbm.at[pl.ds(chunk, 128)], ij_v)

    rows_i = []
    rows_j = []
    for g in range(BPW // 16):
        vec_i = ii_v[pl.ds(off + g * 16, 16)]
        vec_j = ij_v[pl.ds(off + g * 16, 16)]
        for l in range(16):
            rows_i.append(vec_i[l])
            rows_j.append(vec_j[l])

    def fetch(row):
        r_i = rows_i[row]
        r_j = rows_j[row]
        sl_i = pl.multiple_of(
            lax.shift_right_logical(r_i, 7) * 128, 128)
        sl_j = pl.multiple_of(
            lax.shift_right_logical(r_j, 7) * 128, 128)
        buf = row % NBUF
        cps = (
            pltpu.async_copy(
                embT_i.at[:, pl.ds(sl_i, 128)], si_v.at[buf], sem),
            pltpu.async_copy(
                embT_j.at[:, pl.ds(sl_j, 128)], sj_v.at[buf], sem),
            pltpu.async_copy(
                biT.at[:, pl.ds(sl_i, 128)], sbi_v.at[buf], sem),
            pltpu.async_copy(
                bjT.at[:, pl.ds(sl_j, 128)], sbj_v.at[buf], sem),
        )
        return cps

    lane = lax.iota(jnp.int32, 16)
    zeros = jnp.zeros((16,), jnp.float32)

    inflight = [fetch(r) for r in range(NBUF)]
    for row in range(BPW):
        for cp in inflight[row % NBUF]:
            cp.wait()
        buf = row % NBUF
        col_i = jnp.broadcast_to(lax.rem(rows_i[row], 128), (16,))
        col_j = jnp.broadcast_to(lax.rem(rows_j[row], 128), (16,))
        for c in range(D // 16):
            crange = lane + (c * 16)
            vi = plsc.load_gather(si_v.at[buf], [crange, col_i])
            vj = plsc.load_gather(sj_v.at[buf], [crange, col_j])
            p_v[row, pl.ds(c * 16, 16)] = vi * vj
        bvi = plsc.load_gather(sbi_v.at[buf], [jnp.zeros((16,), jnp.int32),
                                               col_i])
        bvj = plsc.load_gather(sbj_v.at[buf], [jnp.zeros((16,), jnp.int32),
                                               col_j])
        p_v[row, pl.ds(D, 16)] = jnp.where(lane == 0, bvi + bvj, zeros)
        if row + NBUF < BPW:
            inflight[row % NBUF] = fetch(row + NBUF)

    pltpu.sync_copy(p_v, packed_out.at[pl.ds(base, BPW)])


_sc_kernel = functools.partial(
    pl.kernel,
    out_type=jax.ShapeDtypeStruct((B, PK), jnp.float32),
    mesh=plsc.VectorSubcoreMesh(core_axis_name="c", subcore_axis_name="s"),
    compiler_params=pltpu.CompilerParams(
        needs_layout_passes=False, use_tc_tiling_on_sc=True),
    scratch_types=[
        pltpu.VMEM((128,), jnp.int32),
        pltpu.VMEM((128,), jnp.int32),
        pltpu.VMEM((NBUF, D, 128), jnp.float32),
        pltpu.VMEM((NBUF, D, 128), jnp.float32),
        pltpu.VMEM((NBUF, 1, 128), jnp.float32),
        pltpu.VMEM((NBUF, 1, 128), jnp.float32),
        pltpu.VMEM((BPW, PK), jnp.float32),
        pltpu.SemaphoreType.DMA,
    ],
)(_sc_gather)


ROW_BLK = 128


def _tc_outer(xij_ref, packed_ref, out_ref):
    xf = xij_ref[...].astype(jnp.float32)
    t = jnp.log(xf)
    fx = jnp.where(xf >= X_MAX, jnp.float32(1.0),
                   jnp.exp(ALPHA * jnp.log(xf * (1.0 / X_MAX))))
    s = jnp.sum(packed_ref[:, :D + 16], axis=1, keepdims=True)
    diff = s - t
    res = fx * diff * diff
    out_ref[...] = res[:, None, :]


def kernel(x, emb_i, emb_j, bi, bj):
    idx_i = x[:, 0]
    idx_j = x[:, 1]
    xij2 = x[:, 2].reshape(1, B)

    packed = _sc_kernel(idx_i, idx_j, emb_i.T, emb_j.T, bi.T, bj.T)

    out2 = pl.pallas_call(
        _tc_outer,
        grid=(B // ROW_BLK,),
        in_specs=[
            pl.BlockSpec((1, B), lambda i: (0, 0)),
            pl.BlockSpec((ROW_BLK, PK), lambda i: (i, 0)),
        ],
        out_specs=pl.BlockSpec((ROW_BLK, 1, B), lambda i: (i, 0, 0)),
        out_shape=jax.ShapeDtypeStruct((B, 1, B), jnp.float32),
    )(xij2, packed)

    return out2
